# baseline (speedup 1.0000x reference)
import functools
import math

import jax
import jax.numpy as jnp
from jax import lax
from jax.experimental import pallas as pl
from jax.experimental.pallas import tpu as pltpu
from jax.experimental.pallas import tpu_sc as plsc

D_MODEL = 64
SCALE = math.sqrt(D_MODEL)

NUM_CORES = 2
NUM_SUBCORES = 16
NUM_WORKERS = NUM_CORES * NUM_SUBCORES

CHUNK = 512
GATHERS = CHUNK // 128


def _sc_embed(x2, table, pe2, seq_len, batch):
    n_rows = seq_len * batch
    n_per_w = n_rows // NUM_WORKERS
    n_chunks = n_per_w // CHUNK
    dblk, bblk = D_MODEL // 8, batch // 128

    mesh = plsc.VectorSubcoreMesh(
        core_axis_name="c", subcore_axis_name="s",
        num_cores=NUM_CORES, num_subcores=NUM_SUBCORES,
    )

    @functools.partial(
        pl.kernel,
        mesh=mesh,
        compiler_params=pltpu.CompilerParams(
            use_tc_tiling_on_sc=False, needs_layout_passes=False
        ),
        out_type=jax.ShapeDtypeStruct(
            (seq_len, dblk, bblk, 8, 128), jnp.float32
        ),
        scratch_types=[
            pltpu.VMEM((GATHERS, 128), jnp.int32),
            pltpu.VMEM((CHUNK, D_MODEL), jnp.float32),
            pltpu.VMEM((8, CHUNK // 128, 8, 128), jnp.float32),
            pltpu.VMEM((8, D_MODEL), jnp.float32),
            pltpu.SemaphoreType.DMA,
        ],
    )
    def sc_kernel(x_hbm, tbl_hbm, pe_hbm, out_hbm, idx_v, rows_v, out_t, pe_v,
                  sem):
        wid = lax.axis_index("s") * NUM_CORES + lax.axis_index("c")
        base = wid * n_per_w
        lane = lax.iota(jnp.int32, 16)
        dblk_vecs = [2 * j + lane // 8 for j in range(4)]
        dsub_vec = lane % 8

        def chunk_body(c, carry):
            row0 = pl.multiple_of(base + c * CHUNK, CHUNK)
            s_pos = row0 // batch
            b0 = row0 - s_pos * batch
            bb0 = pl.multiple_of(b0 // 128, CHUNK // 128)
            pltpu.sync_copy(
                x_hbm.at[pl.ds(pl.multiple_of(row0 // 128, CHUNK // 128), GATHERS)], idx_v
            )
            pltpu.sync_copy(
                pe_hbm.at[pl.ds(pl.multiple_of((s_pos // 8) * 8, 8), 8)], pe_v
            )
            s_sub = s_pos % 8
            copies = [
                pltpu.async_copy(
                    tbl_hbm.at[idx_v.at[j]],
                    rows_v.at[pl.ds(j * 128, 128)],
                    sem,
                )
                for j in range(GATHERS)
            ]
            for cp in copies:
                cp.wait()
            pe_regs = [pe_v[s_sub, pl.ds(16 * j, 16)] for j in range(4)]

            def col_body(bb, bcarry):
                bb_vec = jnp.full((16,), bb, jnp.int32)

                def row_body(r2, rcarry):
                    r2_vec = jnp.full((16,), r2, jnp.int32)
                    for j in range(4):
                        v = (
                            rows_v[bb * 128 + r2, pl.ds(16 * j, 16)] * SCALE
                            + pe_regs[j]
                        )
                        plsc.store_scatter(
                            out_t, [dblk_vecs[j], bb_vec, dsub_vec, r2_vec], v
                        )
                    return rcarry

                lax.fori_loop(0, 128, row_body, 0, unroll=4)
                return bcarry

            lax.fori_loop(0, CHUNK // 128, col_body, 0)
            pltpu.sync_copy(
                out_t, out_hbm.at[s_pos, :, pl.ds(bb0, CHUNK // 128)]
            )
            return carry

        lax.fori_loop(0, n_chunks, chunk_body, 0)

    return sc_kernel(x2, table, pe2)


def kernel(x, table, pe):
    seq_len, batch = x.shape
    n_rows = seq_len * batch
    x2 = x.reshape(n_rows // 128, 128).astype(jnp.int32)
    pe2 = pe.reshape(-1, D_MODEL)
    out5 = _sc_embed(x2, table, pe2, seq_len, batch)
    out = out5.transpose(0, 2, 4, 1, 3).reshape(seq_len, batch, D_MODEL)
    return out


# speedup vs baseline: 1.4345x; 1.4345x over previous
import functools
import math

import jax
import jax.numpy as jnp
from jax import lax
from jax.experimental import pallas as pl
from jax.experimental.pallas import tpu as pltpu
from jax.experimental.pallas import tpu_sc as plsc

D_MODEL = 64
SCALE = math.sqrt(D_MODEL)

NUM_CORES = 2
NUM_SUBCORES = 16
NUM_WORKERS = NUM_CORES * NUM_SUBCORES

CHUNK = 512
GATHERS = CHUNK // 128
CWORDS = CHUNK * D_MODEL // 8


def _sc_embed(x2, table, pe2, seq_len, batch):
    n_rows = seq_len * batch
    n_per_w = n_rows // NUM_WORKERS
    n_chunks = n_per_w // CHUNK

    mesh = plsc.VectorSubcoreMesh(
        core_axis_name="c", subcore_axis_name="s",
        num_cores=NUM_CORES, num_subcores=NUM_SUBCORES,
    )

    @functools.partial(
        pl.kernel,
        mesh=mesh,
        compiler_params=pltpu.CompilerParams(
            use_tc_tiling_on_sc=False, needs_layout_passes=False
        ),
        out_type=jax.ShapeDtypeStruct(
            (seq_len, D_MODEL // 8, batch // 128, 8, 128), jnp.float32
        ),
        scratch_types=[
            pltpu.VMEM((GATHERS, 128), jnp.int32),
            pltpu.VMEM((CHUNK, D_MODEL), jnp.float32),
            pltpu.VMEM((8, 5, 8, 129), jnp.float32),
            pltpu.VMEM((8, D_MODEL), jnp.float32),
            pltpu.SemaphoreType.DMA,
        ],
    )
    def sc_kernel(x_hbm, tbl_hbm, pe_hbm, out_hbm, idx_v, rows_v, out_t, pe_v,
                  sem):
        wid = lax.axis_index("s") * NUM_CORES + lax.axis_index("c")
        base = wid * n_per_w
        lane = lax.iota(jnp.int32, 16)
        dblk_vecs = [2 * j + lane // 8 for j in range(4)]
        dsub_vec = lane % 8

        def chunk_body(c, carry):
            row0 = pl.multiple_of(base + c * CHUNK, CHUNK)
            s_pos = row0 // batch
            b0 = row0 - s_pos * batch
            bb0 = pl.multiple_of(b0 // 128, CHUNK // 128)
            pltpu.sync_copy(
                x_hbm.at[
                    pl.ds(pl.multiple_of(row0 // 128, CHUNK // 128), GATHERS)
                ],
                idx_v,
            )
            pltpu.sync_copy(
                pe_hbm.at[pl.ds(pl.multiple_of((s_pos // 8) * 8, 8), 8)], pe_v
            )
            s_sub = s_pos % 8
            copies = [
                pltpu.async_copy(
                    tbl_hbm.at[idx_v.at[j]],
                    rows_v.at[pl.ds(j * 128, 128)],
                    sem,
                )
                for j in range(GATHERS)
            ]
            for cp in copies:
                cp.wait()
            pe_regs = [pe_v[s_sub, pl.ds(16 * j, 16)] for j in range(4)]

            def col_body(bb, bcarry):
                bb_vec = jnp.full((16,), bb, jnp.int32)

                def row_body(r2, rcarry):
                    r2_vec = jnp.full((16,), r2, jnp.int32)
                    for j in range(4):
                        v = (
                            rows_v[bb * 128 + r2, pl.ds(16 * j, 16)] * SCALE
                            + pe_regs[j]
                        )
                        plsc.store_scatter(
                            out_t, [dblk_vecs[j], bb_vec, dsub_vec, r2_vec], v
                        )
                    return rcarry

                lax.fori_loop(0, 128, row_body, 0, unroll=8)
                return bcarry

            lax.fori_loop(0, CHUNK // 128, col_body, 0)
            out_copies = [
                pltpu.async_copy(
                    out_t.at[dblk, pl.ds(0, 4), :, pl.ds(0, 128)],
                    out_hbm.at[s_pos, dblk, pl.ds(bb0, CHUNK // 128)],
                    sem,
                )
                for dblk in range(8)
            ]
            for cp in out_copies:
                cp.wait()
            return carry

        lax.fori_loop(0, n_chunks, chunk_body, 0)

    return sc_kernel(x2, table, pe2)


def kernel(x, table, pe):
    seq_len, batch = x.shape
    n_rows = seq_len * batch
    x2 = x.reshape(n_rows // 128, 128).astype(jnp.int32)
    pe2 = pe.reshape(-1, D_MODEL)
    out5 = _sc_embed(x2, table, pe2, seq_len, batch)
    out = out5.transpose(0, 2, 4, 1, 3).reshape(seq_len, batch, D_MODEL)
    return out


# speedup vs baseline: 2.1767x; 1.5174x over previous
import functools
import math

import jax
import jax.numpy as jnp
from jax import lax
from jax.experimental import pallas as pl
from jax.experimental.pallas import tpu as pltpu
from jax.experimental.pallas import tpu_sc as plsc

D_MODEL = 64
SCALE = math.sqrt(D_MODEL)

NUM_CORES = 2
NUM_SUBCORES = 16
NUM_WORKERS = NUM_CORES * NUM_SUBCORES

CHUNK = 512
GATHERS = CHUNK // 128
CWORDS = CHUNK * D_MODEL // 8


def _sc_embed(x2, table, pe2, seq_len, batch):
    n_rows = seq_len * batch
    n_per_w = n_rows // NUM_WORKERS
    n_chunks = n_per_w // CHUNK

    mesh = plsc.VectorSubcoreMesh(
        core_axis_name="c", subcore_axis_name="s",
        num_cores=NUM_CORES, num_subcores=NUM_SUBCORES,
    )

    @functools.partial(
        pl.kernel,
        mesh=mesh,
        compiler_params=pltpu.CompilerParams(
            use_tc_tiling_on_sc=False, needs_layout_passes=False
        ),
        out_type=jax.ShapeDtypeStruct(
            (seq_len, D_MODEL // 8, batch // 128, 8, 128), jnp.float32
        ),
        scratch_types=[
            pltpu.VMEM((GATHERS, 128), jnp.int32),
            pltpu.VMEM((CHUNK, D_MODEL), jnp.float32),
            pltpu.VMEM((8, 5, 8, 129), jnp.float32),
            pltpu.VMEM((8, D_MODEL), jnp.float32),
            pltpu.SemaphoreType.DMA,
        ],
    )
    def sc_kernel(x_hbm, tbl_hbm, pe_hbm, out_hbm, idx_v, rows_v, out_t, pe_v,
                  sem):
        wid = lax.axis_index("s") * NUM_CORES + lax.axis_index("c")
        base = wid * n_per_w
        lane = lax.iota(jnp.int32, 16)
        dblk_vecs = [2 * j + lane // 8 for j in range(4)]
        dsub_vec = lane % 8

        def chunk_body(c, carry):
            row0 = pl.multiple_of(base + c * CHUNK, CHUNK)
            s_pos = row0 // batch
            b0 = row0 - s_pos * batch
            bb0 = pl.multiple_of(b0 // 128, CHUNK // 128)
            pltpu.sync_copy(
                x_hbm.at[
                    pl.ds(pl.multiple_of(row0 // 128, CHUNK // 128), GATHERS)
                ],
                idx_v,
            )
            pltpu.sync_copy(
                pe_hbm.at[pl.ds(pl.multiple_of((s_pos // 8) * 8, 8), 8)], pe_v
            )
            s_sub = s_pos % 8
            copies = [
                pltpu.async_copy(
                    tbl_hbm.at[idx_v.at[j]],
                    rows_v.at[pl.ds(j * 128, 128)],
                    sem,
                )
                for j in range(GATHERS)
            ]
            for cp in copies:
                cp.wait()
            pe_regs = [pe_v[s_sub, pl.ds(16 * j, 16)] for j in range(4)]

            def col_body(bb, bcarry):
                bb_vec = jnp.full((16,), bb, jnp.int32)

                @plsc.parallel_loop(0, 128, unroll=8)
                def row_body(r2):
                    r2_vec = jnp.full((16,), r2, jnp.int32)
                    for j in range(4):
                        v = (
                            rows_v[bb * 128 + r2, pl.ds(16 * j, 16)] * SCALE
                            + pe_regs[j]
                        )
                        plsc.store_scatter(
                            out_t, [dblk_vecs[j], bb_vec, dsub_vec, r2_vec], v
                        )

                return bcarry

            lax.fori_loop(0, CHUNK // 128, col_body, 0)
            out_copies = [
                pltpu.async_copy(
                    out_t.at[dblk, pl.ds(0, 4), :, pl.ds(0, 128)],
                    out_hbm.at[s_pos, dblk, pl.ds(bb0, CHUNK // 128)],
                    sem,
                )
                for dblk in range(8)
            ]
            for cp in out_copies:
                cp.wait()
            return carry

        lax.fori_loop(0, n_chunks, chunk_body, 0)

    return sc_kernel(x2, table, pe2)


def kernel(x, table, pe):
    seq_len, batch = x.shape
    n_rows = seq_len * batch
    x2 = x.reshape(n_rows // 128, 128).astype(jnp.int32)
    pe2 = pe.reshape(-1, D_MODEL)
    out5 = _sc_embed(x2, table, pe2, seq_len, batch)
    out = out5.transpose(0, 2, 4, 1, 3).reshape(seq_len, batch, D_MODEL)
    return out


# speedup vs baseline: 2.9499x; 1.3552x over previous
import functools
import math

import jax
import jax.numpy as jnp
from jax import lax
from jax.experimental import pallas as pl
from jax.experimental.pallas import tpu as pltpu
from jax.experimental.pallas import tpu_sc as plsc

D_MODEL = 64
SCALE = math.sqrt(D_MODEL)

NUM_CORES = 2
NUM_SUBCORES = 16
NUM_WORKERS = NUM_CORES * NUM_SUBCORES

CHUNK = 512
GATHERS = CHUNK // 128
CWORDS = CHUNK * D_MODEL // 8


def _sc_embed(x2, table, pe2, seq_len, batch):
    n_rows = seq_len * batch
    n_per_w = n_rows // NUM_WORKERS
    n_chunks = n_per_w // CHUNK

    mesh = plsc.VectorSubcoreMesh(
        core_axis_name="c", subcore_axis_name="s",
        num_cores=NUM_CORES, num_subcores=NUM_SUBCORES,
    )

    @functools.partial(
        pl.kernel,
        mesh=mesh,
        compiler_params=pltpu.CompilerParams(
            use_tc_tiling_on_sc=False, needs_layout_passes=False
        ),
        out_type=jax.ShapeDtypeStruct(
            (seq_len, D_MODEL // 8, batch // 128, 8, 128), jnp.float32
        ),
        scratch_types=[
            pltpu.VMEM((GATHERS, 128), jnp.int32),
            pltpu.VMEM((CHUNK, D_MODEL), jnp.float32),
            pltpu.VMEM((8, 5, 8, 129), jnp.float32),
            pltpu.VMEM((8, D_MODEL), jnp.float32),
            pltpu.SemaphoreType.DMA,
        ],
    )
    def sc_kernel(x_hbm, tbl_hbm, pe_hbm, out_hbm, idx_v, rows_v, out_t, pe_v,
                  sem):
        wid = lax.axis_index("s") * NUM_CORES + lax.axis_index("c")
        base = wid * n_per_w
        lane = lax.iota(jnp.int32, 16)
        dblk_vecs = [2 * j + lane // 8 for j in range(4)]
        dsub_vec = lane % 8

        def chunk_body(c, carry):
            row0 = pl.multiple_of(base + c * CHUNK, CHUNK)
            s_pos = row0 // batch
            b0 = row0 - s_pos * batch
            bb0 = pl.multiple_of(b0 // 128, CHUNK // 128)
            pltpu.sync_copy(
                x_hbm.at[
                    pl.ds(pl.multiple_of(row0 // 128, CHUNK // 128), GATHERS)
                ],
                idx_v,
            )
            pltpu.sync_copy(
                pe_hbm.at[pl.ds(pl.multiple_of((s_pos // 8) * 8, 8), 8)], pe_v
            )
            s_sub = s_pos % 8
            for a in range(GATHERS):
                for k in range(8):
                    vv = idx_v[a, pl.ds(16 * k, 16)]
                    r = vv & 4095
                    idx_v[a, pl.ds(16 * k, 16)] = (
                        (vv - r) + ((r & 2047) << 1) + (r >> 11)
                    )
            copies = [
                pltpu.async_copy(
                    tbl_hbm.at[idx_v.at[j]],
                    rows_v.at[pl.ds(j * 128, 128)],
                    sem,
                )
                for j in range(GATHERS)
            ]
            for cp in copies:
                cp.wait()
            pe_regs = [pe_v[s_sub, pl.ds(16 * j, 16)] for j in range(4)]

            def col_body(bb, bcarry):
                bb_vec = jnp.full((16,), bb, jnp.int32)

                @plsc.parallel_loop(0, 128, unroll=8)
                def row_body(r2):
                    r2_vec = jnp.full((16,), r2, jnp.int32)
                    for j in range(4):
                        v = (
                            rows_v[bb * 128 + r2, pl.ds(16 * j, 16)]
                            + pe_regs[j]
                        )
                        plsc.store_scatter(
                            out_t, [dblk_vecs[j], bb_vec, dsub_vec, r2_vec], v
                        )

                return bcarry

            lax.fori_loop(0, CHUNK // 128, col_body, 0)
            out_copies = [
                pltpu.async_copy(
                    out_t.at[dblk, pl.ds(0, 4), :, pl.ds(0, 128)],
                    out_hbm.at[s_pos, dblk, pl.ds(bb0, CHUNK // 128)],
                    sem,
                )
                for dblk in range(8)
            ]
            for cp in out_copies:
                cp.wait()
            return carry

        lax.fori_loop(0, n_chunks, chunk_body, 0)

    return sc_kernel(x2, table, pe2)


def _tc_relayout(table):
    v_rows, d = table.shape
    bv = 2048
    grid = (v_rows + 2 * bv - 1) // (2 * bv)

    def body(a_ref, b_ref, o_ref):
        o_ref[...] = (
            jnp.concatenate([a_ref[...].T, b_ref[...].T], axis=1) * SCALE
        )

    return pl.pallas_call(
        body,
        grid=(grid,),
        in_specs=[
            pl.BlockSpec((d, bv), lambda i: (0, 2 * i)),
            pl.BlockSpec(
                (d, bv),
                lambda i: (0, jnp.minimum(2 * i + 1, (v_rows - 1) // bv)),
            ),
        ],
        out_specs=pl.BlockSpec((bv, 2 * d), lambda i: (i, 0)),
        out_shape=jax.ShapeDtypeStruct((grid * bv, 2 * d), jnp.float32),
    )(table.T, table.T)


def kernel(x, table, pe):
    seq_len, batch = x.shape
    n_rows = seq_len * batch
    x2 = x.reshape(n_rows // 128, 128).astype(jnp.int32)
    pe2 = pe.reshape(-1, D_MODEL)
    packed = _tc_relayout(table)
    table_lin = packed.reshape(packed.shape[0] * 2, D_MODEL)
    out5 = _sc_embed(x2, table_lin, pe2, seq_len, batch)
    out = out5.transpose(0, 2, 4, 1, 3).reshape(seq_len, batch, D_MODEL)
    return out


# speedup vs baseline: 3.3963x; 1.1513x over previous
import functools
import math

import jax
import jax.numpy as jnp
from jax import lax
from jax.experimental import pallas as pl
from jax.experimental.pallas import tpu as pltpu
from jax.experimental.pallas import tpu_sc as plsc

D_MODEL = 64
SCALE = math.sqrt(D_MODEL)

NUM_CORES = 2
NUM_SUBCORES = 16
NUM_WORKERS = NUM_CORES * NUM_SUBCORES

CHUNK = 512
GATHERS = CHUNK // 128
CWORDS = CHUNK * D_MODEL // 8


def _sc_embed(x2, table, pe2, seq_len, batch):
    n_rows = seq_len * batch
    n_per_w = n_rows // NUM_WORKERS
    n_chunks = n_per_w // CHUNK

    mesh = plsc.VectorSubcoreMesh(
        core_axis_name="c", subcore_axis_name="s",
        num_cores=NUM_CORES, num_subcores=NUM_SUBCORES,
    )

    @functools.partial(
        pl.kernel,
        mesh=mesh,
        compiler_params=pltpu.CompilerParams(
            use_tc_tiling_on_sc=False, needs_layout_passes=False
        ),
        out_type=jax.ShapeDtypeStruct(
            (seq_len, D_MODEL // 8, batch // 128, 8, 128), jnp.float32
        ),
        scratch_types=[
            pltpu.VMEM((2, GATHERS, 128), jnp.int32),
            pltpu.VMEM((2, CHUNK, D_MODEL), jnp.float32),
            pltpu.VMEM((8, 5, 8, 129), jnp.float32),
            pltpu.VMEM((8, D_MODEL), jnp.float32),
            pltpu.SemaphoreType.DMA,
            pltpu.SemaphoreType.DMA,
            pltpu.SemaphoreType.DMA,
        ],
    )
    def sc_kernel(x_hbm, tbl_hbm, pe_hbm, out_hbm, idx_v, rows_v, out_t, pe_v,
                  gsem0, gsem1, osem):
        wid = lax.axis_index("s") * NUM_CORES + lax.axis_index("c")
        base = wid * n_per_w
        lane = lax.iota(jnp.int32, 16)
        gsems = [gsem0, gsem1]
        dblk_vecs = [2 * j + lane // 8 for j in range(4)]
        dsub_vec = lane % 8

        def chunk_coords(c):
            row0 = pl.multiple_of(base + c * CHUNK, CHUNK)
            s_pos = row0 // batch
            b0 = row0 - s_pos * batch
            bb0 = pl.multiple_of(b0 // 128, CHUNK // 128)
            return row0, s_pos, bb0

        def fire_gathers(c, buf):
            row0, _, _ = chunk_coords(c)
            pltpu.sync_copy(
                x_hbm.at[
                    pl.ds(pl.multiple_of(row0 // 128, CHUNK // 128), GATHERS)
                ],
                idx_v.at[buf],
            )
            for a in range(GATHERS):
                for k in range(8):
                    vv = idx_v[buf, a, pl.ds(16 * k, 16)]
                    r = vv & 4095
                    idx_v[buf, a, pl.ds(16 * k, 16)] = (
                        (vv - r) + ((r & 2047) << 1) + (r >> 11)
                    )
            for j in range(GATHERS):
                pltpu.async_copy(
                    tbl_hbm.at[idx_v.at[buf, j]],
                    rows_v.at[buf, pl.ds(j * 128, 128)],
                    gsems[buf],
                )

        def drain_gathers(buf):
            for j in range(GATHERS):
                pltpu.make_async_copy(
                    tbl_hbm.at[pl.ds(0, 128)],
                    rows_v.at[buf, pl.ds(j * 128, 128)],
                    gsems[buf],
                ).wait()

        def drain_out():
            pltpu.make_async_copy(
                out_t.at[:, pl.ds(0, 4), :, pl.ds(0, 128)],
                out_hbm.at[0, :, pl.ds(0, CHUNK // 128)],
                osem,
            ).wait()

        def compute_and_fire_out(c, buf):
            _, s_pos, bb0 = chunk_coords(c)
            pltpu.sync_copy(
                pe_hbm.at[pl.ds(pl.multiple_of((s_pos // 8) * 8, 8), 8)], pe_v
            )
            s_sub = s_pos % 8
            pe_regs = [pe_v[s_sub, pl.ds(16 * j, 16)] for j in range(4)]

            def col_body(bb, bcarry):
                bb_vec = jnp.full((16,), bb, jnp.int32)

                @plsc.parallel_loop(0, 128, unroll=8)
                def row_body(r2):
                    r2_vec = jnp.full((16,), r2, jnp.int32)
                    for j in range(4):
                        v = (
                            rows_v[buf, bb * 128 + r2, pl.ds(16 * j, 16)]
                            + pe_regs[j]
                        )
                        plsc.store_scatter(
                            out_t, [dblk_vecs[j], bb_vec, dsub_vec, r2_vec], v
                        )

                return bcarry

            lax.fori_loop(0, CHUNK // 128, col_body, 0)
            pltpu.async_copy(
                out_t.at[:, pl.ds(0, 4), :, pl.ds(0, 128)],
                out_hbm.at[s_pos, :, pl.ds(bb0, CHUNK // 128)],
                osem,
            )

        fire_gathers(0, 0)

        def pair_body(t, carry):
            c0 = 2 * t
            fire_gathers(c0 + 1, 1)
            drain_gathers(0)

            @pl.when(t > 0)
            def _():
                drain_out()

            compute_and_fire_out(c0, 0)
            fire_gathers(jnp.minimum(c0 + 2, n_chunks - 1), 0)
            drain_gathers(1)
            drain_out()
            compute_and_fire_out(c0 + 1, 1)
            return carry

        lax.fori_loop(0, n_chunks // 2, pair_body, 0)
        drain_out()
        drain_gathers(0)

    return sc_kernel(x2, table, pe2)


def _tc_relayout(table):
    v_rows, d = table.shape
    bv = 2048
    grid = (v_rows + 2 * bv - 1) // (2 * bv)

    def body(a_ref, b_ref, o_ref):
        o_ref[...] = (
            jnp.concatenate([a_ref[...].T, b_ref[...].T], axis=1) * SCALE
        )

    return pl.pallas_call(
        body,
        grid=(grid,),
        in_specs=[
            pl.BlockSpec((d, bv), lambda i: (0, 2 * i)),
            pl.BlockSpec(
                (d, bv),
                lambda i: (0, jnp.minimum(2 * i + 1, (v_rows - 1) // bv)),
            ),
        ],
        out_specs=pl.BlockSpec((bv, 2 * d), lambda i: (i, 0)),
        out_shape=jax.ShapeDtypeStruct((grid * bv, 2 * d), jnp.float32),
    )(table.T, table.T)


def kernel(x, table, pe):
    seq_len, batch = x.shape
    n_rows = seq_len * batch
    x2 = x.reshape(n_rows // 128, 128).astype(jnp.int32)
    pe2 = pe.reshape(-1, D_MODEL)
    packed = _tc_relayout(table)
    table_lin = packed.reshape(packed.shape[0] * 2, D_MODEL)
    out5 = _sc_embed(x2, table_lin, pe2, seq_len, batch)
    out = out5.transpose(0, 2, 4, 1, 3).reshape(seq_len, batch, D_MODEL)
    return out


# speedup vs baseline: 3.5544x; 1.0465x over previous
import functools
import math

import jax
import jax.numpy as jnp
from jax import lax
from jax.experimental import pallas as pl
from jax.experimental.pallas import tpu as pltpu
from jax.experimental.pallas import tpu_sc as plsc

D_MODEL = 64
SCALE = math.sqrt(D_MODEL)

NUM_CORES = 2
NUM_SUBCORES = 16
NUM_WORKERS = NUM_CORES * NUM_SUBCORES

CHUNK = 512
GATHERS = CHUNK // 128
CWORDS = CHUNK * D_MODEL // 8


def _sc_embed(x2, table, pe2, seq_len, batch):
    n_rows = seq_len * batch
    n_per_w = n_rows // NUM_WORKERS
    n_chunks = n_per_w // CHUNK

    mesh = plsc.VectorSubcoreMesh(
        core_axis_name="c", subcore_axis_name="s",
        num_cores=NUM_CORES, num_subcores=NUM_SUBCORES,
    )

    @functools.partial(
        pl.kernel,
        mesh=mesh,
        compiler_params=pltpu.CompilerParams(
            use_tc_tiling_on_sc=False, needs_layout_passes=False
        ),
        out_type=jax.ShapeDtypeStruct(
            (seq_len, D_MODEL // 8, batch // 128, 8, 128), jnp.float32
        ),
        scratch_types=[
            pltpu.VMEM((2, GATHERS, 128), jnp.int32),
            pltpu.VMEM((2, CHUNK, D_MODEL), jnp.float32),
            pltpu.VMEM((8, 5, 8, 129), jnp.float32),
            pltpu.VMEM((8, D_MODEL), jnp.float32),
            pltpu.SemaphoreType.DMA,
            pltpu.SemaphoreType.DMA,
            pltpu.SemaphoreType.DMA,
        ],
    )
    def sc_kernel(x_hbm, tbl_hbm, pe_hbm, out_hbm, idx_v, rows_v, out_t, pe_v,
                  gsem0, gsem1, osem):
        wid = lax.axis_index("s") * NUM_CORES + lax.axis_index("c")
        base = wid * n_per_w
        lane = lax.iota(jnp.int32, 16)
        gsems = [gsem0, gsem1]
        dblk_vecs = [2 * j + lane // 8 for j in range(4)]
        dsub_vec = lane % 8

        def chunk_coords(c):
            row0 = pl.multiple_of(base + c * CHUNK, CHUNK)
            s_pos = row0 // batch
            b0 = row0 - s_pos * batch
            bb0 = pl.multiple_of(b0 // 128, CHUNK // 128)
            return row0, s_pos, bb0

        def fire_gathers(c, buf):
            row0, _, _ = chunk_coords(c)
            pltpu.sync_copy(
                x_hbm.at[
                    pl.ds(pl.multiple_of(row0 // 128, CHUNK // 128), GATHERS)
                ],
                idx_v.at[buf],
            )
            for a in range(GATHERS):
                for k in range(8):
                    vv = idx_v[buf, a, pl.ds(16 * k, 16)]
                    r = vv & 4095
                    idx_v[buf, a, pl.ds(16 * k, 16)] = (
                        (vv - r) + ((r & 2047) << 1) + (r >> 11)
                    )
            for j in range(GATHERS):
                pltpu.async_copy(
                    tbl_hbm.at[idx_v.at[buf, j]],
                    rows_v.at[buf, pl.ds(j * 128, 128)],
                    gsems[buf],
                )

        def drain_gathers(buf):
            for j in range(GATHERS):
                pltpu.make_async_copy(
                    tbl_hbm.at[pl.ds(0, 128)],
                    rows_v.at[buf, pl.ds(j * 128, 128)],
                    gsems[buf],
                ).wait()

        def drain_out():
            pltpu.make_async_copy(
                out_t.at[:, pl.ds(0, 4), :, pl.ds(0, 128)],
                out_hbm.at[0, :, pl.ds(0, CHUNK // 128)],
                osem,
            ).wait()

        def compute_and_fire_out(c, buf):
            _, s_pos, bb0 = chunk_coords(c)

            @pl.when(jnp.logical_or(c == 0, bb0 == 0))
            def _():
                pltpu.sync_copy(
                    pe_hbm.at[pl.ds(pl.multiple_of((s_pos // 8) * 8, 8), 8)],
                    pe_v,
                )
            s_sub = s_pos % 8
            pe_regs = [pe_v[s_sub, pl.ds(16 * j, 16)] for j in range(4)]

            def col_body(bb, bcarry):
                bb_vec = jnp.full((16,), bb, jnp.int32)

                @plsc.parallel_loop(0, 128, unroll=8)
                def row_body(r2):
                    r2_vec = jnp.full((16,), r2, jnp.int32)
                    for j in range(4):
                        v = (
                            rows_v[buf, bb * 128 + r2, pl.ds(16 * j, 16)]
                            + pe_regs[j]
                        )
                        plsc.store_scatter(
                            out_t, [dblk_vecs[j], bb_vec, dsub_vec, r2_vec], v
                        )

                return bcarry

            lax.fori_loop(0, CHUNK // 128, col_body, 0)
            pltpu.async_copy(
                out_t.at[:, pl.ds(0, 4), :, pl.ds(0, 128)],
                out_hbm.at[s_pos, :, pl.ds(bb0, CHUNK // 128)],
                osem,
            )

        fire_gathers(0, 0)

        def pair_body(t, carry):
            c0 = 2 * t
            fire_gathers(c0 + 1, 1)
            drain_gathers(0)

            @pl.when(t > 0)
            def _():
                drain_out()

            compute_and_fire_out(c0, 0)
            fire_gathers(jnp.minimum(c0 + 2, n_chunks - 1), 0)
            drain_gathers(1)
            drain_out()
            compute_and_fire_out(c0 + 1, 1)
            return carry

        lax.fori_loop(0, n_chunks // 2, pair_body, 0)
        drain_out()
        drain_gathers(0)

    return sc_kernel(x2, table, pe2)


def _tc_relayout(table):
    v_rows, d = table.shape
    bv = 2048
    grid = (v_rows + 2 * bv - 1) // (2 * bv)

    def body(a_ref, b_ref, o_ref):
        o_ref[...] = (
            jnp.concatenate([a_ref[...].T, b_ref[...].T], axis=1) * SCALE
        )

    return pl.pallas_call(
        body,
        grid=(grid,),
        in_specs=[
            pl.BlockSpec((d, bv), lambda i: (0, 2 * i)),
            pl.BlockSpec(
                (d, bv),
                lambda i: (0, jnp.minimum(2 * i + 1, (v_rows - 1) // bv)),
            ),
        ],
        out_specs=pl.BlockSpec((bv, 2 * d), lambda i: (i, 0)),
        out_shape=jax.ShapeDtypeStruct((grid * bv, 2 * d), jnp.float32),
    )(table.T, table.T)


def kernel(x, table, pe):
    seq_len, batch = x.shape
    n_rows = seq_len * batch
    x2 = x.reshape(n_rows // 128, 128).astype(jnp.int32)
    pe2 = pe.reshape(-1, D_MODEL)
    packed = _tc_relayout(table)
    table_lin = packed.reshape(packed.shape[0] * 2, D_MODEL)
    out5 = _sc_embed(x2, table_lin, pe2, seq_len, batch)
    out = out5.transpose(0, 2, 4, 1, 3).reshape(seq_len, batch, D_MODEL)
    return out


# speedup vs baseline: 4.0264x; 1.1328x over previous
import functools
import math

import jax
import jax.numpy as jnp
from jax import lax
from jax.experimental import pallas as pl
from jax.experimental.pallas import tpu as pltpu
from jax.experimental.pallas import tpu_sc as plsc

D_MODEL = 64
SCALE = math.sqrt(D_MODEL)

NUM_CORES = 2
NUM_SUBCORES = 16
NUM_WORKERS = NUM_CORES * NUM_SUBCORES

CHUNK = 512
GATHERS = CHUNK // 128
CWORDS = CHUNK * D_MODEL // 8


def _sc_embed(x2, table, pe2, seq_len, batch):
    n_rows = seq_len * batch
    n_per_w = n_rows // NUM_WORKERS
    n_chunks = n_per_w // CHUNK

    mesh = plsc.VectorSubcoreMesh(
        core_axis_name="c", subcore_axis_name="s",
        num_cores=NUM_CORES, num_subcores=NUM_SUBCORES,
    )

    @functools.partial(
        pl.kernel,
        mesh=mesh,
        compiler_params=pltpu.CompilerParams(
            use_tc_tiling_on_sc=False, needs_layout_passes=False
        ),
        out_type=jax.ShapeDtypeStruct(
            (seq_len, D_MODEL // 8, batch // 128, 8, 128), jnp.float32
        ),
        scratch_types=[
            pltpu.VMEM((2, GATHERS, 128), jnp.int32),
            pltpu.VMEM((2, CHUNK, D_MODEL), jnp.float32),
            pltpu.VMEM((8, 5, 8, 129), jnp.float32),
            pltpu.VMEM((8, D_MODEL), jnp.float32),
            pltpu.SemaphoreType.DMA,
            pltpu.SemaphoreType.DMA,
            pltpu.SemaphoreType.DMA,
        ],
    )
    def sc_kernel(x_hbm, tbl_hbm, pe_hbm, out_hbm, idx_v, rows_v, out_t, pe_v,
                  gsem0, gsem1, osem):
        wid = lax.axis_index("s") * NUM_CORES + lax.axis_index("c")
        base = wid * n_per_w
        lane = lax.iota(jnp.int32, 16)
        gsems = [gsem0, gsem1]
        dblk_vecs = [2 * j + lane // 8 for j in range(4)]
        dsub_vec = lane % 8

        def chunk_coords(c):
            row0 = pl.multiple_of(base + c * CHUNK, CHUNK)
            s_pos = row0 // batch
            b0 = row0 - s_pos * batch
            bb0 = pl.multiple_of(b0 // 128, CHUNK // 128)
            return row0, s_pos, bb0

        def fire_gathers(c, buf):
            row0, _, _ = chunk_coords(c)
            pltpu.sync_copy(
                x_hbm.at[
                    pl.ds(pl.multiple_of(row0 // 128, CHUNK // 128), GATHERS)
                ],
                idx_v.at[buf],
            )
            for a in range(GATHERS):
                for k in range(8):
                    vv = idx_v[buf, a, pl.ds(16 * k, 16)]
                    r = vv & 8191
                    idx_v[buf, a, pl.ds(16 * k, 16)] = (
                        (vv - r) + ((r & 4095) << 1) + (r >> 12)
                    )
            for j in range(GATHERS):
                pltpu.async_copy(
                    tbl_hbm.at[idx_v.at[buf, j]],
                    rows_v.at[buf, pl.ds(j * 128, 128)],
                    gsems[buf],
                )

        def drain_gathers(buf):
            for j in range(GATHERS):
                pltpu.make_async_copy(
                    tbl_hbm.at[pl.ds(0, 128)],
                    rows_v.at[buf, pl.ds(j * 128, 128)],
                    gsems[buf],
                ).wait()

        def drain_out():
            pltpu.make_async_copy(
                out_t.at[:, pl.ds(0, 4), :, pl.ds(0, 128)],
                out_hbm.at[0, :, pl.ds(0, CHUNK // 128)],
                osem,
            ).wait()

        def compute_and_fire_out(c, buf):
            _, s_pos, bb0 = chunk_coords(c)

            @pl.when(jnp.logical_or(c == 0, bb0 == 0))
            def _():
                pltpu.sync_copy(
                    pe_hbm.at[pl.ds(pl.multiple_of((s_pos // 8) * 8, 8), 8)],
                    pe_v,
                )
            s_sub = s_pos % 8
            pe_regs = [pe_v[s_sub, pl.ds(16 * j, 16)] for j in range(4)]

            def col_body(bb, bcarry):
                bb_vec = jnp.full((16,), bb, jnp.int32)

                @plsc.parallel_loop(0, 128, unroll=8)
                def row_body(r2):
                    r2_vec = jnp.full((16,), r2, jnp.int32)
                    for j in range(4):
                        v = (
                            rows_v[buf, bb * 128 + r2, pl.ds(16 * j, 16)]
                            + pe_regs[j]
                        )
                        plsc.store_scatter(
                            out_t, [dblk_vecs[j], bb_vec, dsub_vec, r2_vec], v
                        )

                return bcarry

            lax.fori_loop(0, CHUNK // 128, col_body, 0)
            pltpu.async_copy(
                out_t.at[:, pl.ds(0, 4), :, pl.ds(0, 128)],
                out_hbm.at[s_pos, :, pl.ds(bb0, CHUNK // 128)],
                osem,
            )

        fire_gathers(0, 0)

        def pair_body(t, carry):
            c0 = 2 * t
            fire_gathers(c0 + 1, 1)
            drain_gathers(0)

            @pl.when(t > 0)
            def _():
                drain_out()

            compute_and_fire_out(c0, 0)
            fire_gathers(jnp.minimum(c0 + 2, n_chunks - 1), 0)
            drain_gathers(1)
            drain_out()
            compute_and_fire_out(c0 + 1, 1)
            return carry

        lax.fori_loop(0, n_chunks // 2, pair_body, 0)
        drain_out()
        drain_gathers(0)

    return sc_kernel(x2, table, pe2)


def _tc_relayout(table):
    v_rows, d = table.shape
    bv = 4096
    grid = (v_rows + 2 * bv - 1) // (2 * bv)

    def body(a_ref, b_ref, o_ref):
        o_ref[...] = (
            jnp.concatenate([a_ref[...].T, b_ref[...].T], axis=1) * SCALE
        )

    return pl.pallas_call(
        body,
        grid=(grid,),
        in_specs=[
            pl.BlockSpec((d, bv), lambda i: (0, 2 * i)),
            pl.BlockSpec(
                (d, bv),
                lambda i: (0, jnp.minimum(2 * i + 1, (v_rows - 1) // bv)),
            ),
        ],
        out_specs=pl.BlockSpec((bv, 2 * d), lambda i: (i, 0)),
        out_shape=jax.ShapeDtypeStruct((grid * bv, 2 * d), jnp.float32),
    )(table.T, table.T)


def kernel(x, table, pe):
    seq_len, batch = x.shape
    n_rows = seq_len * batch
    x2 = x.reshape(n_rows // 128, 128).astype(jnp.int32)
    pe2 = pe.reshape(-1, D_MODEL)
    packed = _tc_relayout(table)
    table_lin = packed.reshape(packed.shape[0] * 2, D_MODEL)
    out5 = _sc_embed(x2, table_lin, pe2, seq_len, batch)
    out = out5.transpose(0, 2, 4, 1, 3).reshape(seq_len, batch, D_MODEL)
    return out


# speedup vs baseline: 4.2894x; 1.0653x over previous
import functools
import math

import jax
import jax.numpy as jnp
from jax import lax
from jax.experimental import pallas as pl
from jax.experimental.pallas import tpu as pltpu
from jax.experimental.pallas import tpu_sc as plsc

D_MODEL = 64
SCALE = math.sqrt(D_MODEL)

NUM_CORES = 2
NUM_SUBCORES = 16
NUM_WORKERS = NUM_CORES * NUM_SUBCORES

CHUNK = 512
GATHERS = CHUNK // 128
CWORDS = CHUNK * D_MODEL // 8


def _sc_embed(x2, table, pe2, seq_len, batch):
    n_rows = seq_len * batch
    n_per_w = n_rows // NUM_WORKERS
    n_chunks = n_per_w // CHUNK

    mesh = plsc.VectorSubcoreMesh(
        core_axis_name="c", subcore_axis_name="s",
        num_cores=NUM_CORES, num_subcores=NUM_SUBCORES,
    )

    @functools.partial(
        pl.kernel,
        mesh=mesh,
        compiler_params=pltpu.CompilerParams(
            use_tc_tiling_on_sc=False, needs_layout_passes=False
        ),
        out_type=jax.ShapeDtypeStruct(
            (seq_len, D_MODEL // 8, batch // 128, 8, 128), jnp.float32
        ),
        scratch_types=[
            pltpu.VMEM((2, GATHERS, 128), jnp.int32),
            pltpu.VMEM((2, CHUNK, D_MODEL), jnp.float32),
            pltpu.VMEM((8, 5, 8, 129), jnp.float32),
            pltpu.VMEM((8, D_MODEL), jnp.float32),
            pltpu.SemaphoreType.DMA,
            pltpu.SemaphoreType.DMA,
            pltpu.SemaphoreType.DMA,
        ],
    )
    def sc_kernel(x_hbm, tbl_hbm, pe_hbm, out_hbm, idx_v, rows_v, out_t, pe_v,
                  gsem0, gsem1, osem):
        wid = lax.axis_index("s") * NUM_CORES + lax.axis_index("c")
        base = wid * n_per_w
        lane = lax.iota(jnp.int32, 16)
        gsems = [gsem0, gsem1]
        dblk_vecs = [2 * j + lane // 8 for j in range(4)]
        dsub_vec = lane % 8

        def chunk_coords(c):
            row0 = pl.multiple_of(base + c * CHUNK, CHUNK)
            s_pos = row0 // batch
            b0 = row0 - s_pos * batch
            bb0 = pl.multiple_of(b0 // 128, CHUNK // 128)
            return row0, s_pos, bb0

        def fire_gathers(c, buf):
            row0, _, _ = chunk_coords(c)
            pltpu.sync_copy(
                x_hbm.at[
                    pl.ds(pl.multiple_of(row0 // 128, CHUNK // 128), GATHERS)
                ],
                idx_v.at[buf],
            )
            for a in range(GATHERS):
                for k in range(8):
                    vv = idx_v[buf, a, pl.ds(16 * k, 16)]
                    r = vv & 16383
                    idx_v[buf, a, pl.ds(16 * k, 16)] = (
                        (vv - r) + ((r & 8191) << 1) + (r >> 13)
                    )
            for j in range(GATHERS):
                pltpu.async_copy(
                    tbl_hbm.at[idx_v.at[buf, j]],
                    rows_v.at[buf, pl.ds(j * 128, 128)],
                    gsems[buf],
                )

        def drain_gathers(buf):
            for j in range(GATHERS):
                pltpu.make_async_copy(
                    tbl_hbm.at[pl.ds(0, 128)],
                    rows_v.at[buf, pl.ds(j * 128, 128)],
                    gsems[buf],
                ).wait()

        def drain_out():
            pltpu.make_async_copy(
                out_t.at[:, pl.ds(0, 4), :, pl.ds(0, 128)],
                out_hbm.at[0, :, pl.ds(0, CHUNK // 128)],
                osem,
            ).wait()

        def compute_and_fire_out(c, buf):
            _, s_pos, bb0 = chunk_coords(c)

            @pl.when(jnp.logical_or(c == 0, bb0 == 0))
            def _():
                pltpu.sync_copy(
                    pe_hbm.at[pl.ds(pl.multiple_of((s_pos // 8) * 8, 8), 8)],
                    pe_v,
                )
            s_sub = s_pos % 8
            pe_regs = [pe_v[s_sub, pl.ds(16 * j, 16)] for j in range(4)]

            def col_body(bb, bcarry):
                bb_vec = jnp.full((16,), bb, jnp.int32)

                @plsc.parallel_loop(0, 128, unroll=8)
                def row_body(r2):
                    r2_vec = jnp.full((16,), r2, jnp.int32)
                    for j in range(4):
                        v = (
                            rows_v[buf, bb * 128 + r2, pl.ds(16 * j, 16)]
                            + pe_regs[j]
                        )
                        plsc.store_scatter(
                            out_t, [dblk_vecs[j], bb_vec, dsub_vec, r2_vec], v
                        )

                return bcarry

            lax.fori_loop(0, CHUNK // 128, col_body, 0)
            pltpu.async_copy(
                out_t.at[:, pl.ds(0, 4), :, pl.ds(0, 128)],
                out_hbm.at[s_pos, :, pl.ds(bb0, CHUNK // 128)],
                osem,
            )

        fire_gathers(0, 0)

        def pair_body(t, carry):
            c0 = 2 * t
            fire_gathers(c0 + 1, 1)
            drain_gathers(0)

            @pl.when(t > 0)
            def _():
                drain_out()

            compute_and_fire_out(c0, 0)
            fire_gathers(jnp.minimum(c0 + 2, n_chunks - 1), 0)
            drain_gathers(1)
            drain_out()
            compute_and_fire_out(c0 + 1, 1)
            return carry

        lax.fori_loop(0, n_chunks // 2, pair_body, 0)
        drain_out()
        drain_gathers(0)

    return sc_kernel(x2, table, pe2)


def _tc_relayout(table):
    v_rows, d = table.shape
    bv = 8192
    grid = (v_rows + 2 * bv - 1) // (2 * bv)

    def body(a_ref, b_ref, o_ref):
        o_ref[...] = (
            jnp.concatenate([a_ref[...].T, b_ref[...].T], axis=1) * SCALE
        )

    return pl.pallas_call(
        body,
        grid=(grid,),
        in_specs=[
            pl.BlockSpec((d, bv), lambda i: (0, 2 * i)),
            pl.BlockSpec(
                (d, bv),
                lambda i: (0, jnp.minimum(2 * i + 1, (v_rows - 1) // bv)),
            ),
        ],
        out_specs=pl.BlockSpec((bv, 2 * d), lambda i: (i, 0)),
        out_shape=jax.ShapeDtypeStruct((grid * bv, 2 * d), jnp.float32),
    )(table.T, table.T)


def kernel(x, table, pe):
    seq_len, batch = x.shape
    n_rows = seq_len * batch
    x2 = x.reshape(n_rows // 128, 128).astype(jnp.int32)
    pe2 = pe.reshape(-1, D_MODEL)
    packed = _tc_relayout(table)
    table_lin = packed.reshape(packed.shape[0] * 2, D_MODEL)
    out5 = _sc_embed(x2, table_lin, pe2, seq_len, batch)
    out = out5.transpose(0, 2, 4, 1, 3).reshape(seq_len, batch, D_MODEL)
    return out


# speedup vs baseline: 4.4247x; 1.0316x over previous
import functools
import math

import jax
import jax.numpy as jnp
from jax import lax
from jax.experimental import pallas as pl
from jax.experimental.pallas import tpu as pltpu
from jax.experimental.pallas import tpu_sc as plsc

D_MODEL = 64
SCALE = math.sqrt(D_MODEL)

NUM_CORES = 2
NUM_SUBCORES = 16
NUM_WORKERS = NUM_CORES * NUM_SUBCORES

CHUNK = 512
GATHERS = CHUNK // 128
CWORDS = CHUNK * D_MODEL // 8


def _sc_embed(x2, table, pe2, seq_len, batch):
    n_rows = seq_len * batch
    n_per_w = n_rows // NUM_WORKERS
    n_chunks = n_per_w // CHUNK

    mesh = plsc.VectorSubcoreMesh(
        core_axis_name="c", subcore_axis_name="s",
        num_cores=NUM_CORES, num_subcores=NUM_SUBCORES,
    )

    @functools.partial(
        pl.kernel,
        mesh=mesh,
        compiler_params=pltpu.CompilerParams(
            use_tc_tiling_on_sc=False, needs_layout_passes=False
        ),
        out_type=jax.ShapeDtypeStruct(
            (seq_len, D_MODEL // 8, batch // 128, 8, 128), jnp.float32
        ),
        scratch_types=[
            pltpu.VMEM((2, GATHERS, 128), jnp.int32),
            pltpu.VMEM((2, CHUNK, D_MODEL), jnp.float32),
            pltpu.VMEM((8, 5, 8, 129), jnp.float32),
            pltpu.VMEM((8, D_MODEL), jnp.float32),
            pltpu.SemaphoreType.DMA,
            pltpu.SemaphoreType.DMA,
            pltpu.SemaphoreType.DMA,
        ],
    )
    def sc_kernel(x_hbm, tbl_hbm, pe_hbm, out_hbm, idx_v, rows_v, out_t, pe_v,
                  gsem0, gsem1, osem):
        wid = lax.axis_index("s") * NUM_CORES + lax.axis_index("c")
        base = wid * n_per_w
        lane = lax.iota(jnp.int32, 16)
        gsems = [gsem0, gsem1]
        dblk_vecs = [2 * j + lane // 8 for j in range(4)]
        dsub_vec = lane % 8

        def chunk_coords(c):
            row0 = pl.multiple_of(base + c * CHUNK, CHUNK)
            s_pos = row0 // batch
            b0 = row0 - s_pos * batch
            bb0 = pl.multiple_of(b0 // 128, CHUNK // 128)
            return row0, s_pos, bb0

        def fire_gathers(c, buf):
            row0, _, _ = chunk_coords(c)
            pltpu.sync_copy(
                x_hbm.at[
                    pl.ds(pl.multiple_of(row0 // 128, CHUNK // 128), GATHERS)
                ],
                idx_v.at[buf],
            )
            for a in range(GATHERS):
                for k in range(8):
                    vv = idx_v[buf, a, pl.ds(16 * k, 16)]
                    r = vv & 32767
                    idx_v[buf, a, pl.ds(16 * k, 16)] = (
                        (vv - r) + ((r & 16383) << 1) + (r >> 14)
                    )
            for j in range(GATHERS):
                pltpu.async_copy(
                    tbl_hbm.at[idx_v.at[buf, j]],
                    rows_v.at[buf, pl.ds(j * 128, 128)],
                    gsems[buf],
                )

        def drain_gathers(buf):
            for j in range(GATHERS):
                pltpu.make_async_copy(
                    tbl_hbm.at[pl.ds(0, 128)],
                    rows_v.at[buf, pl.ds(j * 128, 128)],
                    gsems[buf],
                ).wait()

        def drain_out():
            pltpu.make_async_copy(
                out_t.at[:, pl.ds(0, 4), :, pl.ds(0, 128)],
                out_hbm.at[0, :, pl.ds(0, CHUNK // 128)],
                osem,
            ).wait()

        def compute_and_fire_out(c, buf):
            _, s_pos, bb0 = chunk_coords(c)

            @pl.when(jnp.logical_or(c == 0, bb0 == 0))
            def _():
                pltpu.sync_copy(
                    pe_hbm.at[pl.ds(pl.multiple_of((s_pos // 8) * 8, 8), 8)],
                    pe_v,
                )
            s_sub = s_pos % 8
            pe_regs = [pe_v[s_sub, pl.ds(16 * j, 16)] for j in range(4)]

            def col_body(bb, bcarry):
                bb_vec = jnp.full((16,), bb, jnp.int32)

                @plsc.parallel_loop(0, 128, unroll=8)
                def row_body(r2):
                    r2_vec = jnp.full((16,), r2, jnp.int32)
                    for j in range(4):
                        v = (
                            rows_v[buf, bb * 128 + r2, pl.ds(16 * j, 16)]
                            + pe_regs[j]
                        )
                        plsc.store_scatter(
                            out_t, [dblk_vecs[j], bb_vec, dsub_vec, r2_vec], v
                        )

                return bcarry

            lax.fori_loop(0, CHUNK // 128, col_body, 0)
            pltpu.async_copy(
                out_t.at[:, pl.ds(0, 4), :, pl.ds(0, 128)],
                out_hbm.at[s_pos, :, pl.ds(bb0, CHUNK // 128)],
                osem,
            )

        fire_gathers(0, 0)

        def pair_body(t, carry):
            c0 = 2 * t
            fire_gathers(c0 + 1, 1)
            drain_gathers(0)

            @pl.when(t > 0)
            def _():
                drain_out()

            compute_and_fire_out(c0, 0)
            fire_gathers(jnp.minimum(c0 + 2, n_chunks - 1), 0)
            drain_gathers(1)
            drain_out()
            compute_and_fire_out(c0 + 1, 1)
            return carry

        lax.fori_loop(0, n_chunks // 2, pair_body, 0)
        drain_out()
        drain_gathers(0)

    return sc_kernel(x2, table, pe2)


def _tc_relayout(table):
    v_rows, d = table.shape
    bv = 16384
    grid = (v_rows + 2 * bv - 1) // (2 * bv)

    def body(a_ref, b_ref, o_ref):
        o_ref[...] = (
            jnp.concatenate([a_ref[...].T, b_ref[...].T], axis=1) * SCALE
        )

    return pl.pallas_call(
        body,
        grid=(grid,),
        in_specs=[
            pl.BlockSpec((d, bv), lambda i: (0, 2 * i)),
            pl.BlockSpec(
                (d, bv),
                lambda i: (0, jnp.minimum(2 * i + 1, (v_rows - 1) // bv)),
            ),
        ],
        out_specs=pl.BlockSpec((bv, 2 * d), lambda i: (i, 0)),
        out_shape=jax.ShapeDtypeStruct((grid * bv, 2 * d), jnp.float32),
    )(table.T, table.T)


def kernel(x, table, pe):
    seq_len, batch = x.shape
    n_rows = seq_len * batch
    x2 = x.reshape(n_rows // 128, 128).astype(jnp.int32)
    pe2 = pe.reshape(-1, D_MODEL)
    packed = _tc_relayout(table)
    table_lin = packed.reshape(packed.shape[0] * 2, D_MODEL)
    out5 = _sc_embed(x2, table_lin, pe2, seq_len, batch)
    out = out5.transpose(0, 2, 4, 1, 3).reshape(seq_len, batch, D_MODEL)
    return out
